# baseline (speedup 1.0000x reference)
import functools
import math

import jax
import jax.numpy as jnp
from jax import lax
from jax.experimental import pallas as pl
from jax.experimental.pallas import tpu as pltpu
from jax.experimental.pallas import tpu_sc as plsc

DEPTH = 64
SCALE = math.sqrt(DEPTH)

NC = 2
NS = 16
NW = NC * NS
LANES = 16
BB = 128
NT = 200
NBUF = 2


def _make_lookup():
  mesh = plsc.VectorSubcoreMesh(core_axis_name="c", subcore_axis_name="s")

  @functools.partial(
      pl.kernel,
      mesh=mesh,
      out_type=jax.ShapeDtypeStruct((NT, 8, NW, 8, BB), jnp.float32),
      scratch_types=[
          pltpu.VMEM((BB, NT), jnp.int32),
          [pltpu.VMEM((BB,), jnp.int32) for _ in range(NBUF)],
          [pltpu.VMEM((BB, DEPTH), jnp.float32) for _ in range(NBUF)],
          [pltpu.VMEM((DEPTH, BB), jnp.float32) for _ in range(NBUF)],
          [pltpu.SemaphoreType.DMA for _ in range(NBUF)],
          [pltpu.SemaphoreType.DMA for _ in range(NBUF)],
      ],
      compiler_params=pltpu.CompilerParams(
          use_tc_tiling_on_sc=False, needs_layout_passes=False),
  )
  def lookup(lut_hbm, x_hbm, out_hbm, xs, ibufs, gbufs, obufs, gsems, ssems):
    j = lax.axis_index("s") * NC + lax.axis_index("c")
    pltpu.sync_copy(x_hbm.at[pl.ds(j * BB, BB)], xs)

    iota = lax.iota(jnp.int32, 16)

    def extract_idx(t, b):
      for bb in range(BB // LANES):
        ridx = iota + (bb * LANES)
        cidx = iota * 0 + t
        ibufs[b][pl.ds(bb * LANES, LANES)] = plsc.load_gather(
            xs, [ridx, cidx])

    def gather(b):
      return pltpu.make_async_copy(lut_hbm.at[ibufs[b]], gbufs[b], gsems[b])

    def store_start(t, b):
      for dblk in range(DEPTH // 8):
        pltpu.make_async_copy(
            obufs[b].at[pl.ds(dblk * 8, 8)], out_hbm.at[t, dblk, j],
            ssems[b]).start()

    def store_wait(b):
      for dblk in range(DEPTH // 8):
        pltpu.make_async_copy(
            obufs[b].at[pl.ds(dblk * 8, 8)], out_hbm.at[0, dblk, j],
            ssems[b]).wait()

    for b in range(NBUF):
      extract_idx(b, b)
      gather(b).start()

    def do_pair(g, carry):
      for b in range(NBUF):
        t = g * NBUF + b
        gather(b).wait()

        @pl.when(g > 0)
        def _():
          store_wait(b)

        def col(d, c):
          for bb in range(BB // LANES):
            v = plsc.load_gather(gbufs[b], [iota + bb * LANES, iota * 0 + d])
            obufs[b][d, pl.ds(bb * LANES, LANES)] = v * SCALE
          return c

        lax.fori_loop(0, DEPTH, col, 0)

        @pl.when(t + NBUF < NT)
        def _():
          extract_idx(t + NBUF, b)
          gather(b).start()

        store_start(t, b)
      return carry

    lax.fori_loop(0, NT // NBUF, do_pair, 0)

    for b in range(NBUF):
      store_wait(b)

  return lookup


def kernel(x, lut):
  out5 = _make_lookup()(lut, x.astype(jnp.int32))
  return out5.transpose(2, 4, 0, 1, 3).reshape(4096, 200, 64)


# speedup vs baseline: 1.5443x; 1.5443x over previous
import functools
import math

import jax
import jax.numpy as jnp
from jax import lax
from jax.experimental import pallas as pl
from jax.experimental.pallas import tpu as pltpu
from jax.experimental.pallas import tpu_sc as plsc

DEPTH = 64
SCALE = math.sqrt(DEPTH)

NC = 2
NS = 16
NW = NC * NS
LANES = 16
BB = 128
NT = 200
NBUF = 2


def _make_lookup():
  mesh = plsc.VectorSubcoreMesh(core_axis_name="c", subcore_axis_name="s")

  @functools.partial(
      pl.kernel,
      mesh=mesh,
      out_type=jax.ShapeDtypeStruct((NT, 8, NW, 8, BB), jnp.float32),
      scratch_types=[
          pltpu.VMEM((BB, NT), jnp.int32),
          [pltpu.VMEM((BB,), jnp.int32) for _ in range(NBUF)],
          [pltpu.VMEM((BB, DEPTH), jnp.float32) for _ in range(NBUF)],
          [pltpu.VMEM((DEPTH, BB), jnp.float32) for _ in range(NBUF)],
          [pltpu.SemaphoreType.DMA for _ in range(NBUF)],
          [pltpu.SemaphoreType.DMA for _ in range(NBUF)],
      ],
      compiler_params=pltpu.CompilerParams(
          use_tc_tiling_on_sc=False, needs_layout_passes=False),
  )
  def lookup(lut_hbm, x_hbm, out_hbm, xs, ibufs, gbufs, obufs, gsems, ssems):
    j = lax.axis_index("s") * NC + lax.axis_index("c")
    pltpu.sync_copy(x_hbm.at[pl.ds(j * BB, BB)], xs)

    iota = lax.iota(jnp.int32, 16)

    def extract_idx(t, b):
      for bb in range(BB // LANES):
        ridx = iota + (bb * LANES)
        cidx = iota * 0 + t
        ibufs[b][pl.ds(bb * LANES, LANES)] = plsc.load_gather(
            xs, [ridx, cidx])

    def gather(b):
      return pltpu.make_async_copy(lut_hbm.at[ibufs[b]], gbufs[b], gsems[b])

    def store_start(t, b):
      for dblk in range(DEPTH // 8):
        pltpu.make_async_copy(
            obufs[b].at[pl.ds(dblk * 8, 8)], out_hbm.at[t, dblk, j],
            ssems[b]).start()

    def store_wait(b):
      for dblk in range(DEPTH // 8):
        pltpu.make_async_copy(
            obufs[b].at[pl.ds(dblk * 8, 8)], out_hbm.at[0, dblk, j],
            ssems[b]).wait()

    for b in range(NBUF):
      extract_idx(b, b)
      gather(b).start()

    def do_pair(g, carry):
      for b in range(NBUF):
        t = g * NBUF + b
        gather(b).wait()

        @pl.when(g > 0)
        def _():
          store_wait(b)

        @plsc.parallel_loop(0, DEPTH, unroll=4)
        def col(d):
          for bb in range(BB // LANES):
            v = plsc.load_gather(gbufs[b], [iota + bb * LANES, iota * 0 + d])
            obufs[b][d, pl.ds(bb * LANES, LANES)] = v * SCALE

        @pl.when(t + NBUF < NT)
        def _():
          extract_idx(t + NBUF, b)
          gather(b).start()

        store_start(t, b)
      return carry

    lax.fori_loop(0, NT // NBUF, do_pair, 0)

    for b in range(NBUF):
      store_wait(b)

  return lookup


def kernel(x, lut):
  out5 = _make_lookup()(lut, x.astype(jnp.int32))
  return out5.transpose(2, 4, 0, 1, 3).reshape(4096, 200, 64)
